# baseline (speedup 1.0000x reference)
import functools

import jax
import jax.numpy as jnp
from jax.experimental import pallas as pl
from jax.experimental.pallas import tpu as pltpu


def _fold(op, r):
    return op(r, jnp.roll(r, r.shape[-1] // 2, axis=-1))


def _adj_body(K, x_ref, prob_ref, cond_ref, ent_ref, sm_ref):
    x = x_ref[0]
    R, L = x.shape
    V = 2 * R
    F = L // 2

    s = _fold(jnp.add, jnp.sum(x, axis=0, keepdims=True))
    sm = jnp.clip(x / (s + 1e-20), 0.001, 1.0 - 0.001)
    lg = jnp.log(sm)
    ent_ref[...] = (jnp.sum(-sm * lg) / F).reshape(1, 1, 1)

    t = jnp.full((1, L), 2.0, jnp.float32)
    n = jnp.zeros((1, L), jnp.int32)
    c = jnp.zeros((1, L), jnp.int32)
    for _ in range(K):
        m = _fold(jnp.maximum,
                  jnp.max(jnp.where(sm < t, sm, -1.0), axis=0, keepdims=True))
        n_new = _fold(jnp.add,
                      jnp.sum((sm >= m).astype(jnp.int32), axis=0,
                              keepdims=True))
        upd = n < K
        c = jnp.where(upd, n, c)
        t = jnp.where(upd, m, t)
        n = jnp.where(upd, n_new, n)

    lane = jax.lax.broadcasted_iota(jnp.int32, (R, L), 1)
    iota_v = jax.lax.broadcasted_iota(jnp.int32, (R, L), 0) * 2 + (
        lane >= F).astype(jnp.int32)
    e = K - c
    eq = sm == t
    mask = sm > t
    last = jnp.full((1, L), -1, jnp.int32)
    for i in range(K):
        cand = _fold(jnp.minimum,
                     jnp.min(jnp.where(eq & (iota_v > last), iota_v, V),
                             axis=0, keepdims=True))
        mask = mask | ((iota_v == cand) & (i < e))
        last = cand

    cond = (mask & (sm > 1.0 / (V * K))).astype(jnp.int32)
    prob_ref[0] = jnp.where(cond == 1, lg, 0.0)
    cond_ref[0] = cond
    sm_ref[0] = sm


def kernel(stack_exp):
    B, V, F = stack_exp.shape
    K = 8
    R, L = V // 2, 2 * F
    x2 = stack_exp.reshape(B, R, L)
    slab = pl.BlockSpec((1, R, L), lambda b: (b, 0, 0))
    prob, cond, ent, sm = pl.pallas_call(
        functools.partial(_adj_body, K),
        grid=(B,),
        in_specs=[slab],
        out_specs=[slab, slab,
                   pl.BlockSpec((1, 1, 1), lambda b: (b, 0, 0)), slab],
        out_shape=[
            jax.ShapeDtypeStruct((B, R, L), jnp.float32),
            jax.ShapeDtypeStruct((B, R, L), jnp.int32),
            jax.ShapeDtypeStruct((B, 1, 1), jnp.float32),
            jax.ShapeDtypeStruct((B, R, L), jnp.float32),
        ],
        compiler_params=pltpu.CompilerParams(
            dimension_semantics=("parallel",)),
    )(x2)
    return (prob.reshape(B, V, F), cond.reshape(B, V, F),
            ent.reshape(B), sm.reshape(B, V, F))


# speedup vs baseline: 1.2082x; 1.2082x over previous
import functools

import jax
import jax.numpy as jnp
import numpy as np
from jax.experimental import pallas as pl
from jax.experimental.pallas import tpu as pltpu

_CH = 32


def _fold(op, r):
    return op(r, jnp.roll(r, r.shape[-1] // 2, axis=-1))


def _adj_body(K, x_ref, iota_ref, prob_ref, cond_ref, ent_ref, sm_ref):
    R, L = x_ref.shape[1], x_ref.shape[2]
    V = 2 * R
    F = L // 2
    NCH = R // _CH

    def pass_a(i, carry):
        acc = carry[0]
        ms = list(carry[1:])
        v = x_ref[0, pl.ds(i * _CH, _CH), :]
        acc = acc + v
        cur = v
        for j in range(K):
            hi = jnp.maximum(ms[j], cur)
            cur = jnp.minimum(ms[j], cur)
            ms[j] = hi
        return (acc, *ms)

    init_a = (jnp.zeros((_CH, L), jnp.float32),) + tuple(
        jnp.full((_CH, L), -jnp.inf, jnp.float32) for _ in range(K))
    res_a = jax.lax.fori_loop(0, NCH, pass_a, init_a)
    s = _fold(jnp.add, jnp.sum(res_a[0], axis=0, keepdims=True))
    cand_x = jnp.concatenate(res_a[1:], axis=0)
    sm_cand = jnp.clip(cand_x / (s + 1e-20), 0.001, 1.0 - 0.001)

    t = jnp.full((1, L), 2.0, jnp.float32)
    n = jnp.zeros((1, L), jnp.int32)
    c = jnp.zeros((1, L), jnp.int32)
    for _ in range(K):
        m = _fold(jnp.maximum,
                  jnp.max(jnp.where(sm_cand < t, sm_cand, -1.0), axis=0,
                          keepdims=True))
        n_new = _fold(jnp.add,
                      jnp.sum((sm_cand >= m).astype(jnp.int32), axis=0,
                              keepdims=True))
        upd = n < K
        c = jnp.where(upd, n, c)
        t = jnp.where(upd, m, t)
        n = jnp.where(upd, n_new, n)
    e = K - c

    def pass_b(i, carry):
        ent_acc = carry[0]
        js = list(carry[1:])
        xv = x_ref[0, pl.ds(i * _CH, _CH), :]
        io = iota_ref[pl.ds(i * _CH, _CH), :]
        smv = jnp.clip(xv / (s + 1e-20), 0.001, 1.0 - 0.001)
        sm_ref[0, pl.ds(i * _CH, _CH), :] = smv
        ent_acc = ent_acc - smv * jnp.log(smv)
        cur = jnp.where(smv == t, io, V)
        for j in range(K):
            lo = jnp.minimum(js[j], cur)
            cur = jnp.maximum(js[j], cur)
            js[j] = lo
        return (ent_acc, *js)

    init_b = (jnp.zeros((_CH, L), jnp.float32),) + tuple(
        jnp.full((_CH, L), V, jnp.int32) for _ in range(K))
    res_b = jax.lax.fori_loop(0, NCH, pass_b, init_b)
    ent_ref[...] = (jnp.sum(res_b[0]) / F).reshape(1, 1, 1)
    cand_i = jnp.concatenate(res_b[1:], axis=0)

    s_last = jnp.full((1, L), -1, jnp.int32)
    last = jnp.full((1, L), -1, jnp.int32)
    for i in range(K):
        cnd = _fold(jnp.minimum,
                    jnp.min(jnp.where(cand_i > last, cand_i, V), axis=0,
                            keepdims=True))
        s_last = jnp.where(i < e, cnd, s_last)
        last = cnd

    sm2 = sm_ref[0]
    io2 = iota_ref[...]
    mask = (sm2 > t) | ((sm2 == t) & (io2 <= s_last))
    cond = (mask & (sm2 > 1.0 / (V * K))).astype(jnp.int32)
    cond_ref[0] = cond
    prob_ref[0] = jnp.where(cond == 1, jnp.log(sm2), 0.0)


def kernel(stack_exp):
    B, V, F = stack_exp.shape
    K = 8
    R, L = V // 2, 2 * F
    x2 = stack_exp.reshape(B, R, L)
    iota_v = jnp.asarray(
        2 * np.arange(R, dtype=np.int32)[:, None]
        + (np.arange(L, dtype=np.int32)[None, :] >= F))
    slab = pl.BlockSpec((1, R, L), lambda b: (b, 0, 0))
    prob, cond, ent, sm = pl.pallas_call(
        functools.partial(_adj_body, K),
        grid=(B,),
        in_specs=[slab, pl.BlockSpec((R, L), lambda b: (0, 0))],
        out_specs=[slab, slab,
                   pl.BlockSpec((1, 1, 1), lambda b: (b, 0, 0)), slab],
        out_shape=[
            jax.ShapeDtypeStruct((B, R, L), jnp.float32),
            jax.ShapeDtypeStruct((B, R, L), jnp.int32),
            jax.ShapeDtypeStruct((B, 1, 1), jnp.float32),
            jax.ShapeDtypeStruct((B, R, L), jnp.float32),
        ],
        compiler_params=pltpu.CompilerParams(
            dimension_semantics=("parallel",)),
    )(x2, iota_v)
    return (prob.reshape(B, V, F), cond.reshape(B, V, F),
            ent.reshape(B), sm.reshape(B, V, F))
